# Initial kernel scaffold; baseline (speedup 1.0000x reference)
import jax
import jax.numpy as jnp
from jax.experimental import pallas as pl


def kernel(x, edge_index, edge_attr, lower, upper, W1, b1, U1, ub1, W2, b2, U2, ub2, W3, b3, U3, ub3):
    raise NotImplementedError("write your pallas kernel here")


# speedup vs baseline: 11.2966x; 11.2966x over previous
import functools

import jax
import jax.numpy as jnp
from jax import lax
from jax.experimental import pallas as pl
from jax.experimental.pallas import tpu as pltpu
from jax.experimental.pallas import tpu_sc as plsc

N = 10000
E = 640000
DIM = 16

NP = 10112
NC = 2
NS = 16
NW = NC * NS
K = 128
CHUNKS = 158
EW = K * CHUNKS
EP = EW * NW
ROWS_PT = NP // NS
EBLK = 4096


def _dot(a, b):
    return lax.dot_general(a, b, (((1,), (0,)), ((), ())),
                           preferred_element_type=jnp.float32)


def _edge_c_body(ea_ref, w1_ref, b1_ref, w2_ref, b2_ref, w3_ref, b3_ref,
                 c1_ref, c2_ref, c3_ref):
    ea = ea_ref[...]
    c1_ref[...] = _dot(ea, w1_ref[...]) + b1_ref[...]
    c2_ref[...] = _dot(ea, w2_ref[...]) + b2_ref[...]
    c3_ref[...] = _dot(ea, w3_ref[...]) + b3_ref[...]


def _edge_c(ea_p, w1e, b1, w2e, b2, w3e, b3p):
    nblk = EP // EBLK
    full = lambda shp: pl.BlockSpec(shp, lambda i: (0, 0))
    return pl.pallas_call(
        _edge_c_body,
        grid=(nblk,),
        in_specs=[
            pl.BlockSpec((EBLK, DIM), lambda i: (i, 0)),
            full((DIM, 32)), full((1, 32)),
            full((DIM, 16)), full((1, 16)),
            full((DIM, 16)), full((1, 16)),
        ],
        out_specs=[
            pl.BlockSpec((EBLK, 32), lambda i: (i, 0)),
            pl.BlockSpec((EBLK, 16), lambda i: (i, 0)),
            pl.BlockSpec((EBLK, 16), lambda i: (i, 0)),
        ],
        out_shape=[
            jax.ShapeDtypeStruct((EP, 32), jnp.float32),
            jax.ShapeDtypeStruct((EP, 16), jnp.float32),
            jax.ShapeDtypeStruct((EP, 16), jnp.float32),
        ],
    )(ea_p, w1e, b1.reshape(1, -1), w2e, b2.reshape(1, -1),
      w3e, b3p.reshape(1, -1))


def _node1_body(h_ref, wd_ref, ws_ref, wu_ref, ub_ref, a_ref, b_ref, s_ref):
    h = h_ref[...]
    a_ref[...] = _dot(h, wd_ref[...])
    b_ref[...] = _dot(h, ws_ref[...])
    s_ref[...] = _dot(h, wu_ref[...]) + ub_ref[...]


def _node1(h1p, wd, ws, wu, ub):
    d = wd.shape[1]
    return pl.pallas_call(
        _node1_body,
        out_shape=[jax.ShapeDtypeStruct((NP, d), jnp.float32)] * 3,
    )(h1p, wd, ws, wu, ub.reshape(1, -1))


def _mid_body(agg_ref, sp_ref, wd_ref, ws_ref, wu_ref, ub_ref,
              a_ref, b_ref, s_ref):
    h = agg_ref[0:NP, :] + agg_ref[NP:2 * NP, :] + sp_ref[...]
    a_ref[...] = _dot(h, wd_ref[...])
    b_ref[...] = _dot(h, ws_ref[...])
    s_ref[...] = _dot(h, wu_ref[...]) + ub_ref[...]


def _mid(agg, s_prev, wd, ws, wu, ub):
    d = wd.shape[1]
    return pl.pallas_call(
        _mid_body,
        out_shape=[jax.ShapeDtypeStruct((NP, d), jnp.float32)] * 3,
    )(agg, s_prev, wd, ws, wu, ub.reshape(1, -1))


def _final_body(agg_ref, s_ref, o_ref):
    o_ref[...] = jnp.tanh(agg_ref[0:NP, :] + agg_ref[NP:2 * NP, :] + s_ref[...])


def _final(agg, s3):
    return pl.pallas_call(
        _final_body,
        out_shape=jax.ShapeDtypeStruct((NP, 16), jnp.float32),
    )(agg, s3)


def _make_sc_layer(d):
    cs = d // 16
    mesh = plsc.VectorSubcoreMesh(core_axis_name="c", subcore_axis_name="s")

    @functools.partial(
        pl.kernel,
        mesh=mesh,
        compiler_params=pltpu.CompilerParams(use_tc_tiling_on_sc=False),
        out_type=jax.ShapeDtypeStruct((NC * NP, d), jnp.float32),
        scratch_types=[
            pltpu.VMEM((EW,), jnp.int32),
            pltpu.VMEM((EW,), jnp.int32),
            pltpu.VMEM((2, K), jnp.int32),
            pltpu.VMEM((2, K, d), jnp.float32),
            pltpu.VMEM((2, K, d), jnp.float32),
            pltpu.VMEM((2, K, d), jnp.float32),
            pltpu.VMEM_SHARED((NP, d), jnp.float32),
            pltpu.SemaphoreType.DMA,
            pltpu.SemaphoreType.DMA,
        ],
    )
    def sc_layer(src_hbm, dst_hbm, zero_hbm, a_hbm, b_hbm, c_hbm, out_hbm,
                 srcv, dstv, dsts, av, bv, cv, agg, sem0, sem1):
        cid = lax.axis_index("c")
        sid = lax.axis_index("s")
        wid = sid * NC + cid
        ebase = wid * EW
        row0 = sid * ROWS_PT
        sems = (sem0, sem1)

        pltpu.sync_copy(zero_hbm, agg.at[pl.ds(row0, ROWS_PT)])

        pltpu.sync_copy(src_hbm.at[pl.ds(ebase, EW)], srcv)
        pltpu.sync_copy(dst_hbm.at[pl.ds(ebase, EW)], dstv)

        plsc.subcore_barrier()

        def _issue(slot, g):
            for j in range(K // 16):
                dsts[slot, pl.ds(j * 16, 16)] = dstv[pl.ds(g * K + j * 16, 16)]
            pltpu.async_copy(c_hbm.at[pl.ds(ebase + g * K, K)],
                             cv.at[slot], sems[slot])
            pltpu.async_copy(a_hbm.at[dsts.at[slot]], av.at[slot], sems[slot])
            pltpu.async_copy(b_hbm.at[srcv.at[pl.ds(g * K, K)]],
                             bv.at[slot], sems[slot])

        def _process(slot):
            pltpu.make_async_copy(c_hbm.at[pl.ds(0, K)],
                                  cv.at[slot], sems[slot]).wait()
            pltpu.make_async_copy(a_hbm.at[dsts.at[slot]],
                                  av.at[slot], sems[slot]).wait()
            pltpu.make_async_copy(b_hbm.at[srcv.at[pl.ds(0, K)]],
                                  bv.at[slot], sems[slot]).wait()

            def _row(r, carry):
                for j in range(cs):
                    sl = pl.ds(j * 16, 16)
                    v = av[slot, r, sl] + bv[slot, r, sl] + cv[slot, r, sl]
                    av[slot, r, sl] = jnp.maximum(v, v * 0.01)
                return carry

            lax.fori_loop(0, K, _row, 0)
            pltpu.sync_copy(av.at[slot], agg.at[dsts.at[slot]], add=True)

        _issue(0, 0)
        _issue(1, 1)

        def _pair(it, carry):
            g = it * 2
            _process(0)
            _issue(0, g + 2)
            _process(1)
            _issue(1, g + 3)
            return carry

        lax.fori_loop(0, CHUNKS // 2 - 1, _pair, 0)
        _process(0)
        _process(1)

        plsc.subcore_barrier()
        pltpu.sync_copy(agg.at[pl.ds(row0, ROWS_PT)],
                        out_hbm.at[pl.ds(cid * NP + row0, ROWS_PT)])

    return sc_layer


_sc32 = _make_sc_layer(32)
_sc16 = _make_sc_layer(16)


def kernel(x, edge_index, edge_attr, lower, upper,
           W1, b1, U1, ub1, W2, b2, U2, ub2, W3, b3, U3, ub3):
    f32 = jnp.float32
    pad_e = EP - E
    src_p = jnp.concatenate([edge_index[0],
                             jnp.zeros((pad_e,), jnp.int32)])
    dst_p = jnp.concatenate([edge_index[1],
                             N + (jnp.arange(pad_e, dtype=jnp.int32) % (NP - N))])
    ea_p = jnp.concatenate([edge_attr, jnp.zeros((pad_e, DIM), f32)], axis=0)

    h1 = jnp.concatenate([x, lower, upper], axis=1)
    h1p = jnp.concatenate([h1, jnp.zeros((NP - N, 66), f32)], axis=0)

    W1d, W1s, W1e = W1[:, :66].T, W1[:, 66:132].T, W1[:, 132:].T
    W2d, W2s, W2e = W2[:, :32].T, W2[:, 32:64].T, W2[:, 64:].T
    W3p = jnp.pad(W3, ((0, 9), (0, 0)))
    W3d, W3s, W3e = W3p[:, :16].T, W3p[:, 16:32].T, W3p[:, 32:].T
    b3p = jnp.pad(b3, (0, 9))
    U3t = jnp.pad(U3, ((0, 9), (0, 0))).T
    ub3p = jnp.pad(ub3, (0, 9))

    zero32 = jnp.zeros((ROWS_PT, 32), f32)
    zero16 = jnp.zeros((ROWS_PT, 16), f32)

    C1, C2, C3 = _edge_c(ea_p, W1e, b1, W2e, b2, W3e, b3p)

    A1, B1, S1 = _node1(h1p, W1d, W1s, U1.T, ub1)
    agg1 = _sc32(src_p, dst_p, zero32, A1, B1, C1)
    A2, B2, S2 = _mid(agg1, S1, W2d, W2s, U2.T, ub2)
    agg2 = _sc16(src_p, dst_p, zero16, A2, B2, C2)
    A3, B3, S3 = _mid(agg2, S2, W3d, W3s, U3t, ub3p)
    agg3 = _sc16(src_p, dst_p, zero16, A3, B3, C3)
    out = _final(agg3, S3)
    return out[:N, :7]


# speedup vs baseline: 17.7606x; 1.5722x over previous
import functools

import jax
import jax.numpy as jnp
from jax import lax
from jax.experimental import pallas as pl
from jax.experimental.pallas import tpu as pltpu
from jax.experimental.pallas import tpu_sc as plsc

N = 10000
E = 640000
DIM = 16

NP = 10112
NC = 2
NS = 16
NW = NC * NS
K = 128
EW = E // NW
CHUNKS = EW // K
TAIL = EW - CHUNKS * K
ROWS_PT = NP // NS
EBLK = 1024


def _dot(a, b):
    return lax.dot_general(a, b, (((1,), (0,)), ((), ())),
                           preferred_element_type=jnp.float32)


def _edge_c_body(ea4_ref, ea8_ref, w1_ref, b1_ref, w2_ref, b2_ref,
                 w3_ref, b3_ref, c1_ref, c2_ref, c3_ref):
    c1_ref[...] = _dot(ea4_ref[...], w1_ref[...]) + b1_ref[...]
    ea8 = ea8_ref[...]
    c2_ref[...] = _dot(ea8, w2_ref[...]) + b2_ref[...]
    c3_ref[...] = _dot(ea8, w3_ref[...]) + b3_ref[...]


def _edge_c(ea4, ea8, w1bd, b1t, w2bd, b2t, w3bd, b3t):
    nblk = 125
    full = lambda shp: pl.BlockSpec(shp, lambda i: (0, 0))
    return pl.pallas_call(
        _edge_c_body,
        grid=(nblk,),
        in_specs=[
            pl.BlockSpec((1280, 64), lambda i: (i, 0)),
            pl.BlockSpec((640, 128), lambda i: (i, 0)),
            full((64, 128)), full((1, 128)),
            full((128, 128)), full((1, 128)),
            full((128, 128)), full((1, 128)),
        ],
        out_specs=[
            pl.BlockSpec((1280, 128), lambda i: (i, 0)),
            pl.BlockSpec((640, 128), lambda i: (i, 0)),
            pl.BlockSpec((640, 128), lambda i: (i, 0)),
        ],
        out_shape=[
            jax.ShapeDtypeStruct((E // 4, 128), jnp.float32),
            jax.ShapeDtypeStruct((E // 8, 128), jnp.float32),
            jax.ShapeDtypeStruct((E // 8, 128), jnp.float32),
        ],
    )(ea4, ea8, w1bd, b1t.reshape(1, -1), w2bd, b2t.reshape(1, -1),
      w3bd, b3t.reshape(1, -1))


def _block_diag(w, copies):
    i, o = w.shape
    out = jnp.zeros((copies * i, copies * o), w.dtype)
    for q in range(copies):
        out = out.at[q * i:(q + 1) * i, q * o:(q + 1) * o].set(w)
    return out


def _node1_body(h_ref, wd_ref, ws_ref, wu_ref, ub_ref, a_ref, b_ref, s_ref):
    h = h_ref[...]
    a_ref[...] = _dot(h, wd_ref[...])
    b_ref[...] = _dot(h, ws_ref[...])
    s_ref[...] = _dot(h, wu_ref[...]) + ub_ref[...]


def _node1(h1p, wd, ws, wu, ub):
    d = wd.shape[1]
    return pl.pallas_call(
        _node1_body,
        out_shape=[jax.ShapeDtypeStruct((NP, d), jnp.float32)] * 3,
    )(h1p, wd, ws, wu, ub.reshape(1, -1))


def _mid_body(agg_ref, sp_ref, wd_ref, ws_ref, wu_ref, ub_ref,
              a_ref, b_ref, s_ref):
    h = agg_ref[0:NP, :] + agg_ref[NP:2 * NP, :] + sp_ref[...]
    a_ref[...] = _dot(h, wd_ref[...])
    b_ref[...] = _dot(h, ws_ref[...])
    s_ref[...] = _dot(h, wu_ref[...]) + ub_ref[...]


def _mid(agg, s_prev, wd, ws, wu, ub):
    d = wd.shape[1]
    return pl.pallas_call(
        _mid_body,
        out_shape=[jax.ShapeDtypeStruct((NP, d), jnp.float32)] * 3,
    )(agg, s_prev, wd, ws, wu, ub.reshape(1, -1))


def _final_body(agg_ref, s_ref, o_ref):
    o_ref[...] = jnp.tanh(agg_ref[0:NP, :] + agg_ref[NP:2 * NP, :] + s_ref[...])


def _final(agg, s3):
    return pl.pallas_call(
        _final_body,
        out_shape=jax.ShapeDtypeStruct((NP, 16), jnp.float32),
    )(agg, s3)


def _make_sc_layer(d):
    mesh = plsc.VectorSubcoreMesh(core_axis_name="c", subcore_axis_name="s")
    cpr = 128 // d
    crows = K // cpr
    trows = TAIL // cpr

    @functools.partial(
        pl.kernel,
        mesh=mesh,
        compiler_params=pltpu.CompilerParams(use_tc_tiling_on_sc=False),
        out_type=jax.ShapeDtypeStruct((NC * NP, d), jnp.float32),
        scratch_types=[
            pltpu.VMEM((EW,), jnp.int32),
            pltpu.VMEM((EW,), jnp.int32),
            pltpu.VMEM((2, K), jnp.int32),
            pltpu.VMEM((2, K, d), jnp.float32),
            pltpu.VMEM((2, K, d), jnp.float32),
            pltpu.VMEM((2, crows, 128), jnp.float32),
            pltpu.VMEM((TAIL,), jnp.int32),
            pltpu.VMEM((TAIL, d), jnp.float32),
            pltpu.VMEM((TAIL, d), jnp.float32),
            pltpu.VMEM((trows, 128), jnp.float32),
            pltpu.VMEM_SHARED((NP, d), jnp.float32),
            pltpu.SemaphoreType.DMA,
            pltpu.SemaphoreType.DMA,
            pltpu.SemaphoreType.DMA,
        ],
    )
    def sc_layer(src_hbm, dst_hbm, zero_hbm, a_hbm, b_hbm, c_hbm, out_hbm,
                 srcv, dstv, dsts, av, bv, cv, tdst, tav, tbv, tcv, agg,
                 sem0, sem1, sem2):
        cid = lax.axis_index("c")
        sid = lax.axis_index("s")
        wid = sid * NC + cid
        ebase = wid * EW
        cbase = wid * (EW // cpr)
        row0 = sid * ROWS_PT
        sems = (sem0, sem1)

        pltpu.sync_copy(zero_hbm, agg.at[pl.ds(row0, ROWS_PT)])

        pltpu.sync_copy(src_hbm.at[pl.ds(ebase, EW)], srcv)
        pltpu.sync_copy(dst_hbm.at[pl.ds(ebase, EW)], dstv)

        plsc.subcore_barrier()

        for j in range(TAIL // 16):
            tdst[pl.ds(j * 16, 16)] = dstv[pl.ds(CHUNKS * K + j * 16, 16)]
        pltpu.async_copy(c_hbm.at[pl.ds(cbase + CHUNKS * crows, trows)],
                         tcv, sem2)
        pltpu.async_copy(a_hbm.at[tdst], tav, sem2)
        pltpu.async_copy(b_hbm.at[srcv.at[pl.ds(CHUNKS * K, TAIL)]], tbv, sem2)

        def _issue(slot, g):
            for j in range(K // 16):
                dsts[slot, pl.ds(j * 16, 16)] = dstv[pl.ds(g * K + j * 16, 16)]
            pltpu.async_copy(c_hbm.at[pl.ds(cbase + g * crows, crows)],
                             cv.at[slot], sems[slot])
            pltpu.async_copy(a_hbm.at[dsts.at[slot]], av.at[slot], sems[slot])
            pltpu.async_copy(b_hbm.at[srcv.at[pl.ds(g * K, K)]],
                             bv.at[slot], sems[slot])

        def _process(slot):
            pltpu.make_async_copy(c_hbm.at[pl.ds(0, crows)],
                                  cv.at[slot], sems[slot]).wait()
            pltpu.make_async_copy(a_hbm.at[dsts.at[slot]],
                                  av.at[slot], sems[slot]).wait()
            pltpu.make_async_copy(b_hbm.at[srcv.at[pl.ds(0, K)]],
                                  bv.at[slot], sems[slot]).wait()

            def _row(rr, carry):
                for j in range(8):
                    f = j * 16
                    ar = rr * cpr + f // d
                    asl = pl.ds(f % d, 16)
                    v = (av[slot, ar, asl] + bv[slot, ar, asl]
                         + cv[slot, rr, pl.ds(f, 16)])
                    av[slot, ar, asl] = jnp.maximum(v, v * 0.01)
                return carry

            lax.fori_loop(0, crows, _row, 0)
            pltpu.sync_copy(av.at[slot], agg.at[dsts.at[slot]], add=True)

        _issue(0, 0)
        _issue(1, 1)

        def _pair(it, carry):
            g = it * 2
            _process(0)
            _issue(0, g + 2)
            _process(1)
            _issue(1, g + 3)
            return carry

        lax.fori_loop(0, CHUNKS // 2 - 1, _pair, 0)
        _process(0)
        _process(1)

        pltpu.make_async_copy(c_hbm.at[pl.ds(0, trows)], tcv, sem2).wait()
        pltpu.make_async_copy(a_hbm.at[tdst], tav, sem2).wait()
        pltpu.make_async_copy(b_hbm.at[srcv.at[pl.ds(0, TAIL)]],
                              tbv, sem2).wait()

        def _trow(rr, carry):
            for j in range(8):
                f = j * 16
                ar = rr * cpr + f // d
                asl = pl.ds(f % d, 16)
                v = tav[ar, asl] + tbv[ar, asl] + tcv[rr, pl.ds(f, 16)]
                tav[ar, asl] = jnp.maximum(v, v * 0.01)
            return carry

        lax.fori_loop(0, trows, _trow, 0)
        pltpu.sync_copy(tav, agg.at[tdst], add=True)

        plsc.subcore_barrier()
        pltpu.sync_copy(agg.at[pl.ds(row0, ROWS_PT)],
                        out_hbm.at[pl.ds(cid * NP + row0, ROWS_PT)])

    return sc_layer


_sc32 = _make_sc_layer(32)
_sc16 = _make_sc_layer(16)


def kernel(x, edge_index, edge_attr, lower, upper,
           W1, b1, U1, ub1, W2, b2, U2, ub2, W3, b3, U3, ub3):
    f32 = jnp.float32
    src = edge_index[0]
    dst = edge_index[1]

    h1 = jnp.concatenate([x, lower, upper], axis=1)
    h1p = jnp.concatenate([h1, jnp.zeros((NP - N, 66), f32)], axis=0)

    W1d, W1s, W1e = W1[:, :66].T, W1[:, 66:132].T, W1[:, 132:].T
    W2d, W2s, W2e = W2[:, :32].T, W2[:, 32:64].T, W2[:, 64:].T
    W3p = jnp.pad(W3, ((0, 9), (0, 0)))
    W3d, W3s, W3e = W3p[:, :16].T, W3p[:, 16:32].T, W3p[:, 32:].T
    b3p = jnp.pad(b3, (0, 9))
    U3t = jnp.pad(U3, ((0, 9), (0, 0))).T
    ub3p = jnp.pad(ub3, (0, 9))

    zero32 = jnp.zeros((ROWS_PT, 32), f32)
    zero16 = jnp.zeros((ROWS_PT, 16), f32)

    ea4 = edge_attr.reshape(E // 4, 64)
    ea8 = edge_attr.reshape(E // 8, 128)
    C1, C2, C3 = _edge_c(ea4, ea8,
                         _block_diag(W1e, 4), jnp.tile(b1, 4),
                         _block_diag(W2e, 8), jnp.tile(b2, 8),
                         _block_diag(W3e, 8), jnp.tile(b3p, 8))

    A1, B1, S1 = _node1(h1p, W1d, W1s, U1.T, ub1)
    agg1 = _sc32(src, dst, zero32, A1, B1, C1)
    A2, B2, S2 = _mid(agg1, S1, W2d, W2s, U2.T, ub2)
    agg2 = _sc16(src, dst, zero16, A2, B2, C2)
    A3, B3, S3 = _mid(agg2, S2, W3d, W3s, U3t, ub3p)
    agg3 = _sc16(src, dst, zero16, A3, B3, C3)
    out = _final(agg3, S3)
    return out[:N, :7]


# speedup vs baseline: 18.7640x; 1.0565x over previous
import functools

import jax
import jax.numpy as jnp
from jax import lax
from jax.experimental import pallas as pl
from jax.experimental.pallas import tpu as pltpu
from jax.experimental.pallas import tpu_sc as plsc

N = 10000
E = 640000
DIM = 16

NP = 10112
NC = 2
NS = 16
NW = NC * NS
K = 128
EW = E // NW
CHUNKS = EW // K
TAIL = EW - CHUNKS * K
ROWS_PT = NP // NS
EBLK = 1024


def _dot(a, b):
    return lax.dot_general(a, b, (((1,), (0,)), ((), ())),
                           preferred_element_type=jnp.float32)


def _edge_c1_body(ea8_ref, we_ref, wo_ref, b_ref, ce_ref, co_ref):
    ea8 = ea8_ref[...]
    ce_ref[...] = _dot(ea8, we_ref[...]) + b_ref[...]
    co_ref[...] = _dot(ea8, wo_ref[...]) + b_ref[...]


def _edge_c1(ea8, we, wo, bt):
    nblk = 125
    full = lambda shp: pl.BlockSpec(shp, lambda i: (0, 0))
    return pl.pallas_call(
        _edge_c1_body,
        grid=(nblk,),
        in_specs=[
            pl.BlockSpec((640, 128), lambda i: (i, 0)),
            full((128, 128)), full((128, 128)), full((1, 128)),
        ],
        out_specs=[
            pl.BlockSpec((640, 128), lambda i: (i, 0)),
            pl.BlockSpec((640, 128), lambda i: (i, 0)),
        ],
        out_shape=[
            jax.ShapeDtypeStruct((E // 8, 128), jnp.float32),
            jax.ShapeDtypeStruct((E // 8, 128), jnp.float32),
        ],
    )(ea8, we, wo, bt.reshape(1, -1))


def _edge_c23_body(ea8_ref, w2_ref, b2_ref, w3_ref, b3_ref, c2_ref, c3_ref):
    ea8 = ea8_ref[...]
    c2_ref[...] = _dot(ea8, w2_ref[...]) + b2_ref[...]
    c3_ref[...] = _dot(ea8, w3_ref[...]) + b3_ref[...]


def _edge_c23(ea8, w2bd, b2t, w3bd, b3t):
    nblk = 125
    full = lambda shp: pl.BlockSpec(shp, lambda i: (0, 0))
    return pl.pallas_call(
        _edge_c23_body,
        grid=(nblk,),
        in_specs=[
            pl.BlockSpec((640, 128), lambda i: (i, 0)),
            full((128, 128)), full((1, 128)),
            full((128, 128)), full((1, 128)),
        ],
        out_specs=[
            pl.BlockSpec((640, 128), lambda i: (i, 0)),
            pl.BlockSpec((640, 128), lambda i: (i, 0)),
        ],
        out_shape=[
            jax.ShapeDtypeStruct((E // 8, 128), jnp.float32),
            jax.ShapeDtypeStruct((E // 8, 128), jnp.float32),
        ],
    )(ea8, w2bd, b2t.reshape(1, -1), w3bd, b3t.reshape(1, -1))


def _block_diag(w, copies):
    i, o = w.shape
    out = jnp.zeros((copies * i, copies * o), w.dtype)
    for q in range(copies):
        out = out.at[q * i:(q + 1) * i, q * o:(q + 1) * o].set(w)
    return out


def _block_diag(w, copies):
    i, o = w.shape
    out = jnp.zeros((copies * i, copies * o), w.dtype)
    for q in range(copies):
        out = out.at[q * i:(q + 1) * i, q * o:(q + 1) * o].set(w)
    return out


def _node1_body(h_ref, wd_ref, ws_ref, wu_ref, ub_ref, a_ref, b_ref, s_ref):
    h = h_ref[...]
    a_ref[...] = _dot(h, wd_ref[...])
    b_ref[...] = _dot(h, ws_ref[...])
    s_ref[...] = _dot(h, wu_ref[...]) + ub_ref[...]


def _node1(h1p, wd, ws, wu, ub):
    d = wd.shape[1]
    return pl.pallas_call(
        _node1_body,
        out_shape=[jax.ShapeDtypeStruct((NP, d), jnp.float32)] * 3,
    )(h1p, wd, ws, wu, ub.reshape(1, -1))


def _mid_body(agg_ref, sp_ref, wd_ref, ws_ref, wu_ref, ub_ref,
              a_ref, b_ref, s_ref):
    h = agg_ref[0:NP, :] + agg_ref[NP:2 * NP, :] + sp_ref[...]
    a_ref[...] = _dot(h, wd_ref[...])
    b_ref[...] = _dot(h, ws_ref[...])
    s_ref[...] = _dot(h, wu_ref[...]) + ub_ref[...]


def _mid(agg, s_prev, wd, ws, wu, ub):
    d = wd.shape[1]
    return pl.pallas_call(
        _mid_body,
        out_shape=[jax.ShapeDtypeStruct((NP, d), jnp.float32)] * 3,
    )(agg, s_prev, wd, ws, wu, ub.reshape(1, -1))


def _final_body(agg_ref, s_ref, o_ref):
    o_ref[...] = jnp.tanh(agg_ref[0:NP, :] + agg_ref[NP:2 * NP, :] + s_ref[...])


def _final(agg, s3):
    return pl.pallas_call(
        _final_body,
        out_shape=jax.ShapeDtypeStruct((NP, 16), jnp.float32),
    )(agg, s3)


def _make_sc_layer(d, split_c):
    mesh = plsc.VectorSubcoreMesh(core_axis_name="c", subcore_axis_name="s")
    crows = K // 8
    trows = TAIL // 8
    ncs = 2 if split_c else 1

    @functools.partial(
        pl.kernel,
        mesh=mesh,
        compiler_params=pltpu.CompilerParams(use_tc_tiling_on_sc=False),
        out_type=jax.ShapeDtypeStruct((NC * NP, d), jnp.float32),
        scratch_types=[
            pltpu.VMEM((EW,), jnp.int32),
            pltpu.VMEM((EW,), jnp.int32),
            pltpu.VMEM((2, K), jnp.int32),
            pltpu.VMEM((2, K, d), jnp.float32),
            pltpu.VMEM((2, K, d), jnp.float32),
        ] + [pltpu.VMEM((2, crows, 128), jnp.float32)] * ncs + [
            pltpu.VMEM((TAIL,), jnp.int32),
            pltpu.VMEM((TAIL, d), jnp.float32),
            pltpu.VMEM((TAIL, d), jnp.float32),
        ] + [pltpu.VMEM((trows, 128), jnp.float32)] * ncs + [
            pltpu.VMEM_SHARED((NP, d), jnp.float32),
            pltpu.SemaphoreType.DMA,
            pltpu.SemaphoreType.DMA,
            pltpu.SemaphoreType.DMA,
        ],
    )
    def sc_layer(src_hbm, dst_hbm, zero_hbm, a_hbm, b_hbm, *rest):
        c_hbms = rest[:ncs]
        out_hbm = rest[ncs]
        srcv, dstv, dsts, av, bv = rest[ncs + 1:ncs + 6]
        cvs = rest[ncs + 6:ncs + 6 + ncs]
        tdst, tav, tbv = rest[ncs + 6 + ncs:ncs + 9 + ncs]
        tcvs = rest[ncs + 9 + ncs:ncs + 9 + 2 * ncs]
        agg, sem0, sem1, sem2 = rest[ncs + 9 + 2 * ncs:]

        cid = lax.axis_index("c")
        sid = lax.axis_index("s")
        wid = sid * NC + cid
        ebase = wid * EW
        cbase = wid * (EW // 8)
        row0 = sid * ROWS_PT
        sems = (sem0, sem1)

        pltpu.sync_copy(zero_hbm, agg.at[pl.ds(row0, ROWS_PT)])

        pltpu.sync_copy(src_hbm.at[pl.ds(ebase, EW)], srcv)
        pltpu.sync_copy(dst_hbm.at[pl.ds(ebase, EW)], dstv)

        plsc.subcore_barrier()

        for j in range(TAIL // 16):
            tdst[pl.ds(j * 16, 16)] = dstv[pl.ds(CHUNKS * K + j * 16, 16)]
        for c_hbm, tcv in zip(c_hbms, tcvs):
            pltpu.async_copy(c_hbm.at[pl.ds(cbase + CHUNKS * crows, trows)],
                             tcv, sem2)
        pltpu.async_copy(a_hbm.at[tdst], tav, sem2)
        pltpu.async_copy(b_hbm.at[srcv.at[pl.ds(CHUNKS * K, TAIL)]], tbv, sem2)

        def _compute(a_r, b_r, c_rs, nrows):
            def _row(rr, carry):
                for ci, c_r in enumerate(c_rs):
                    for j in range(8):
                        f = j * 16
                        if split_c:
                            ar = rr * 8 + 4 * ci + j // 2
                            asl = pl.ds((j % 2) * 16, 16)
                        else:
                            ar = rr * 8 + j
                            asl = pl.ds(0, 16)
                        v = a_r[ar, asl] + b_r[ar, asl] + c_r[rr, pl.ds(f, 16)]
                        a_r[ar, asl] = jnp.maximum(v, v * 0.01)
                return carry
            lax.fori_loop(0, nrows, _row, 0)

        def _issue(slot, g):
            for j in range(K // 16):
                dsts[slot, pl.ds(j * 16, 16)] = dstv[pl.ds(g * K + j * 16, 16)]
            for c_hbm, cv in zip(c_hbms, cvs):
                pltpu.async_copy(c_hbm.at[pl.ds(cbase + g * crows, crows)],
                                 cv.at[slot], sems[slot])
            pltpu.async_copy(a_hbm.at[dsts.at[slot]], av.at[slot], sems[slot])
            pltpu.async_copy(b_hbm.at[srcv.at[pl.ds(g * K, K)]],
                             bv.at[slot], sems[slot])

        def _process(slot):
            for c_hbm, cv in zip(c_hbms, cvs):
                pltpu.make_async_copy(c_hbm.at[pl.ds(0, crows)],
                                      cv.at[slot], sems[slot]).wait()
            pltpu.make_async_copy(a_hbm.at[dsts.at[slot]],
                                  av.at[slot], sems[slot]).wait()
            pltpu.make_async_copy(b_hbm.at[srcv.at[pl.ds(0, K)]],
                                  bv.at[slot], sems[slot]).wait()
            _compute(av.at[slot], bv.at[slot],
                     [cv.at[slot] for cv in cvs], crows)
            pltpu.sync_copy(av.at[slot], agg.at[dsts.at[slot]], add=True)

        _issue(0, 0)
        _issue(1, 1)

        def _pair(it, carry):
            g = it * 2
            _process(0)
            _issue(0, g + 2)
            _process(1)
            _issue(1, g + 3)
            return carry

        lax.fori_loop(0, CHUNKS // 2 - 1, _pair, 0)
        _process(0)
        _process(1)

        for c_hbm, tcv in zip(c_hbms, tcvs):
            pltpu.make_async_copy(c_hbm.at[pl.ds(0, trows)], tcv, sem2).wait()
        pltpu.make_async_copy(a_hbm.at[tdst], tav, sem2).wait()
        pltpu.make_async_copy(b_hbm.at[srcv.at[pl.ds(0, TAIL)]],
                              tbv, sem2).wait()
        _compute(tav, tbv, tcvs, trows)
        pltpu.sync_copy(tav, agg.at[tdst], add=True)

        plsc.subcore_barrier()
        pltpu.sync_copy(agg.at[pl.ds(row0, ROWS_PT)],
                        out_hbm.at[pl.ds(cid * NP + row0, ROWS_PT)])

    return sc_layer


_sc32 = _make_sc_layer(32, split_c=True)
_sc16 = _make_sc_layer(16, split_c=False)


def kernel(x, edge_index, edge_attr, lower, upper,
           W1, b1, U1, ub1, W2, b2, U2, ub2, W3, b3, U3, ub3):
    f32 = jnp.float32
    src = edge_index[0]
    dst = edge_index[1]

    h1 = jnp.concatenate([x, lower, upper], axis=1)
    h1p = jnp.concatenate([h1, jnp.zeros((NP - N, 66), f32)], axis=0)

    W1d, W1s, W1e = W1[:, :66].T, W1[:, 66:132].T, W1[:, 132:].T
    W2d, W2s, W2e = W2[:, :32].T, W2[:, 32:64].T, W2[:, 64:].T
    W3p = jnp.pad(W3, ((0, 9), (0, 0)))
    W3d, W3s, W3e = W3p[:, :16].T, W3p[:, 16:32].T, W3p[:, 32:].T
    b3p = jnp.pad(b3, (0, 9))
    U3t = jnp.pad(U3, ((0, 9), (0, 0))).T
    ub3p = jnp.pad(ub3, (0, 9))

    zero32 = jnp.zeros((ROWS_PT, 32), f32)
    zero16 = jnp.zeros((ROWS_PT, 16), f32)

    ea8 = edge_attr.reshape(E // 8, 128)
    w1bd = _block_diag(W1e, 8)
    C1e, C1o = _edge_c1(ea8, w1bd[:, :128], w1bd[:, 128:], jnp.tile(b1, 4))
    C2, C3 = _edge_c23(ea8, _block_diag(W2e, 8), jnp.tile(b2, 8),
                       _block_diag(W3e, 8), jnp.tile(b3p, 8))

    A1, B1, S1 = _node1(h1p, W1d, W1s, U1.T, ub1)
    agg1 = _sc32(src, dst, zero32, A1, B1, C1e, C1o)
    A2, B2, S2 = _mid(agg1, S1, W2d, W2s, U2.T, ub2)
    agg2 = _sc16(src, dst, zero16, A2, B2, C2)
    A3, B3, S3 = _mid(agg2, S2, W3d, W3s, U3t, ub3p)
    agg3 = _sc16(src, dst, zero16, A3, B3, C3)
    out = _final(agg3, S3)
    return out[:N, :7]
